# Initial kernel scaffold; baseline (speedup 1.0000x reference)
import jax
import jax.numpy as jnp
from jax.experimental import pallas as pl


def kernel(x, token_mask, gate_weight, e_bias, gate_projs, up_projs, down_projs, shared_gate, shared_up, shared_down):
    raise NotImplementedError("write your pallas kernel here")


# speedup vs baseline: 1.4867x; 1.4867x over previous
import functools

import jax
import jax.numpy as jnp
from jax import lax
from jax.experimental import pallas as pl
from jax.experimental.pallas import tpu as pltpu

E = 8
TOPK = 2
DIM = 1024
INTER = 512
ROUTE_SCALE = 2.5
T = 2048
BT = 512


def _gate_weights_block(xb, gw, eb):
    scores = jax.nn.sigmoid(
        lax.dot_general(xb, gw, (((1,), (1,)), ((), ())),
                        preferred_element_type=jnp.float32))
    biased = scores + eb
    idx = lax.broadcasted_iota(jnp.int32, scores.shape, 1)
    m1 = jnp.max(biased, axis=1, keepdims=True)
    first1 = jnp.min(jnp.where(biased == m1, idx, E), axis=1, keepdims=True)
    sel1 = idx == first1
    biased2 = jnp.where(sel1, -jnp.inf, biased)
    m2 = jnp.max(biased2, axis=1, keepdims=True)
    first2 = jnp.min(jnp.where(biased2 == m2, idx, E), axis=1, keepdims=True)
    sel = sel1 | (idx == first2)
    picked = jnp.where(sel, scores, 0.0)
    wsum = jnp.sum(picked, axis=1, keepdims=True)
    return picked * (ROUTE_SCALE / wsum)


def _moe_body(x_ref, gwt_ref, eb_ref, gp_ref, up_ref, dp_ref, out_ref):
    e = pl.program_id(1)
    xb = x_ref[...]
    w_full = _gate_weights_block(xb, gwt_ref[...], eb_ref[...])
    idx = lax.broadcasted_iota(jnp.int32, w_full.shape, 1)
    w_sel = jnp.sum(jnp.where(idx == e, w_full, 0.0), axis=1, keepdims=True)
    w = jnp.where(e < E, w_sel, 1.0)

    gp = gp_ref[0]
    up = up_ref[0]
    dp = dp_ref[0]
    g = lax.dot_general(xb, gp, (((1,), (1,)), ((), ())),
                        preferred_element_type=jnp.float32)
    u = lax.dot_general(xb, up, (((1,), (1,)), ((), ())),
                        preferred_element_type=jnp.float32)
    inter = jax.nn.silu(g) * u
    h = lax.dot_general(inter, dp, (((1,), (1,)), ((), ())),
                        preferred_element_type=jnp.float32)

    @pl.when(e == 0)
    def _():
        out_ref[...] = w * h

    @pl.when(e != 0)
    def _():
        out_ref[...] += w * h


@jax.jit
def _moe(x, gate_weight, e_bias, gp_all, up_all, dp_all):
    grid = (T // BT, E + 1)
    return pl.pallas_call(
        _moe_body,
        grid=grid,
        in_specs=[
            pl.BlockSpec((BT, DIM), lambda i, e: (i, 0)),
            pl.BlockSpec((E, DIM), lambda i, e: (0, 0)),
            pl.BlockSpec((1, E), lambda i, e: (0, 0)),
            pl.BlockSpec((1, INTER, DIM), lambda i, e: (e, 0, 0)),
            pl.BlockSpec((1, INTER, DIM), lambda i, e: (e, 0, 0)),
            pl.BlockSpec((1, DIM, INTER), lambda i, e: (e, 0, 0)),
        ],
        out_specs=pl.BlockSpec((BT, DIM), lambda i, e: (i, 0)),
        out_shape=jax.ShapeDtypeStruct((T, DIM), jnp.float32),
        compiler_params=pltpu.CompilerParams(
            dimension_semantics=("parallel", "arbitrary")),
    )(x, gate_weight, e_bias, gp_all, up_all, dp_all)


def kernel(x, token_mask, gate_weight, e_bias, gate_projs, up_projs,
           down_projs, shared_gate, shared_up, shared_down):
    del token_mask
    gp_all = jnp.concatenate([gate_projs, shared_gate[None]], axis=0)
    up_all = jnp.concatenate([up_projs, shared_up[None]], axis=0)
    dp_all = jnp.concatenate([down_projs, shared_down[None]], axis=0)
    return _moe(x, gate_weight, e_bias.reshape(1, E), gp_all, up_all, dp_all)


# speedup vs baseline: 1.5301x; 1.0292x over previous
import functools

import jax
import jax.numpy as jnp
from jax import lax
from jax.experimental import pallas as pl
from jax.experimental.pallas import tpu as pltpu

E = 8
TOPK = 2
DIM = 1024
INTER = 512
ROUTE_SCALE = 2.5
T = 2048
BT = 512


def _gate_weights_block(xb, gw, eb):
    scores = jax.nn.sigmoid(
        lax.dot_general(xb, gw, (((1,), (1,)), ((), ())),
                        preferred_element_type=jnp.float32))
    biased = scores + eb
    idx = lax.broadcasted_iota(jnp.int32, scores.shape, 1)
    m1 = jnp.max(biased, axis=1, keepdims=True)
    first1 = jnp.min(jnp.where(biased == m1, idx, E), axis=1, keepdims=True)
    sel1 = idx == first1
    biased2 = jnp.where(sel1, -jnp.inf, biased)
    m2 = jnp.max(biased2, axis=1, keepdims=True)
    first2 = jnp.min(jnp.where(biased2 == m2, idx, E), axis=1, keepdims=True)
    sel = sel1 | (idx == first2)
    picked = jnp.where(sel, scores, 0.0)
    wsum = jnp.sum(picked, axis=1, keepdims=True)
    return picked * (ROUTE_SCALE / wsum)


def _moe_body(x_ref, gwt_ref, eb_ref, gp_ref, up_ref, dp_ref, out_ref):
    e = pl.program_id(1)
    xb = x_ref[...]
    w_full = _gate_weights_block(xb, gwt_ref[...], eb_ref[...])
    idx = lax.broadcasted_iota(jnp.int32, w_full.shape, 1)
    w_sel = jnp.sum(jnp.where(idx == e, w_full, 0.0), axis=1, keepdims=True)
    w = jnp.where(e < E, w_sel, 1.0)

    xb16 = xb.astype(jnp.bfloat16)
    gp = gp_ref[0]
    up = up_ref[0]
    dp = dp_ref[0]
    g = lax.dot_general(xb16, gp, (((1,), (1,)), ((), ())),
                        preferred_element_type=jnp.float32)
    u = lax.dot_general(xb16, up, (((1,), (1,)), ((), ())),
                        preferred_element_type=jnp.float32)
    inter = (jax.nn.silu(g) * u).astype(jnp.bfloat16)
    h = lax.dot_general(inter, dp, (((1,), (1,)), ((), ())),
                        preferred_element_type=jnp.float32)

    @pl.when(e == 0)
    def _():
        out_ref[...] = w * h

    @pl.when(e != 0)
    def _():
        out_ref[...] += w * h


@jax.jit
def _moe(x, gate_weight, e_bias, gp_all, up_all, dp_all):
    grid = (T // BT, E + 1)
    return pl.pallas_call(
        _moe_body,
        grid=grid,
        in_specs=[
            pl.BlockSpec((BT, DIM), lambda i, e: (i, 0)),
            pl.BlockSpec((E, DIM), lambda i, e: (0, 0)),
            pl.BlockSpec((1, E), lambda i, e: (0, 0)),
            pl.BlockSpec((1, INTER, DIM), lambda i, e: (e, 0, 0)),
            pl.BlockSpec((1, INTER, DIM), lambda i, e: (e, 0, 0)),
            pl.BlockSpec((1, DIM, INTER), lambda i, e: (e, 0, 0)),
        ],
        out_specs=pl.BlockSpec((BT, DIM), lambda i, e: (i, 0)),
        out_shape=jax.ShapeDtypeStruct((T, DIM), jnp.float32),
        compiler_params=pltpu.CompilerParams(
            dimension_semantics=("parallel", "arbitrary")),
    )(x, gate_weight, e_bias, gp_all, up_all, dp_all)


def kernel(x, token_mask, gate_weight, e_bias, gate_projs, up_projs,
           down_projs, shared_gate, shared_up, shared_down):
    del token_mask
    gp_all = jnp.concatenate([gate_projs, shared_gate[None]], axis=0).astype(jnp.bfloat16)
    up_all = jnp.concatenate([up_projs, shared_up[None]], axis=0).astype(jnp.bfloat16)
    dp_all = jnp.concatenate([down_projs, shared_down[None]], axis=0).astype(jnp.bfloat16)
    return _moe(x, gate_weight, e_bias.reshape(1, E), gp_all, up_all, dp_all)


# speedup vs baseline: 1.6771x; 1.0961x over previous
import functools

import jax
import jax.numpy as jnp
from jax import lax
from jax.experimental import pallas as pl
from jax.experimental.pallas import tpu as pltpu

E = 8
TOPK = 2
DIM = 1024
INTER = 512
ROUTE_SCALE = 2.5
T = 2048
BT = 512


def _gate_weights_block(xb, gw, eb):
    scores = jax.nn.sigmoid(
        lax.dot_general(xb, gw, (((1,), (1,)), ((), ())),
                        preferred_element_type=jnp.float32))
    biased = scores + eb
    idx = lax.broadcasted_iota(jnp.int32, scores.shape, 1)
    m1 = jnp.max(biased, axis=1, keepdims=True)
    first1 = jnp.min(jnp.where(biased == m1, idx, E), axis=1, keepdims=True)
    sel1 = idx == first1
    biased2 = jnp.where(sel1, -jnp.inf, biased)
    m2 = jnp.max(biased2, axis=1, keepdims=True)
    first2 = jnp.min(jnp.where(biased2 == m2, idx, E), axis=1, keepdims=True)
    sel = sel1 | (idx == first2)
    picked = jnp.where(sel, scores, 0.0)
    wsum = jnp.sum(picked, axis=1, keepdims=True)
    return picked * (ROUTE_SCALE / wsum)


def _moe_body(x_ref, gwt_ref, eb_ref, gp_ref, up_ref, dp_ref, out_ref, w_scr):
    e = pl.program_id(1)
    xb = x_ref[...]

    @pl.when(e == 0)
    def _():
        w_scr[...] = _gate_weights_block(xb, gwt_ref[...], eb_ref[...])

    w_full = w_scr[...]
    idx = lax.broadcasted_iota(jnp.int32, w_full.shape, 1)
    w_sel = jnp.sum(jnp.where(idx == e, w_full, 0.0), axis=1, keepdims=True)
    w = jnp.where(e < E, w_sel, 1.0)

    xb16 = xb.astype(jnp.bfloat16)
    gp = gp_ref[0]
    up = up_ref[0]
    dp = dp_ref[0]
    g = lax.dot_general(xb16, gp, (((1,), (1,)), ((), ())),
                        preferred_element_type=jnp.float32)
    u = lax.dot_general(xb16, up, (((1,), (1,)), ((), ())),
                        preferred_element_type=jnp.float32)
    inter = (jax.nn.silu(g) * u).astype(jnp.bfloat16)
    h = lax.dot_general(inter, dp, (((1,), (1,)), ((), ())),
                        preferred_element_type=jnp.float32)

    @pl.when(e == 0)
    def _():
        out_ref[...] = w * h

    @pl.when(e != 0)
    def _():
        out_ref[...] += w * h


@jax.jit
def _moe(x, gate_weight, e_bias, gp_all, up_all, dp_all):
    grid = (T // BT, E + 1)
    return pl.pallas_call(
        _moe_body,
        grid=grid,
        in_specs=[
            pl.BlockSpec((BT, DIM), lambda i, e: (i, 0)),
            pl.BlockSpec((E, DIM), lambda i, e: (0, 0)),
            pl.BlockSpec((1, E), lambda i, e: (0, 0)),
            pl.BlockSpec((1, INTER, DIM), lambda i, e: (e, 0, 0)),
            pl.BlockSpec((1, INTER, DIM), lambda i, e: (e, 0, 0)),
            pl.BlockSpec((1, DIM, INTER), lambda i, e: (e, 0, 0)),
        ],
        out_specs=pl.BlockSpec((BT, DIM), lambda i, e: (i, 0)),
        out_shape=jax.ShapeDtypeStruct((T, DIM), jnp.float32),
        scratch_shapes=[pltpu.VMEM((BT, E), jnp.float32)],
        compiler_params=pltpu.CompilerParams(
            dimension_semantics=("parallel", "arbitrary")),
    )(x, gate_weight, e_bias, gp_all, up_all, dp_all)


def kernel(x, token_mask, gate_weight, e_bias, gate_projs, up_projs,
           down_projs, shared_gate, shared_up, shared_down):
    del token_mask
    gp_all = jnp.concatenate([gate_projs, shared_gate[None]], axis=0).astype(jnp.bfloat16)
    up_all = jnp.concatenate([up_projs, shared_up[None]], axis=0).astype(jnp.bfloat16)
    dp_all = jnp.concatenate([down_projs, shared_down[None]], axis=0).astype(jnp.bfloat16)
    return _moe(x, gate_weight, e_bias.reshape(1, E), gp_all, up_all, dp_all)
